# baseline (speedup 1.0000x reference)
import functools

import jax
import jax.numpy as jnp
from jax import lax
from jax.experimental import pallas as pl
from jax.experimental.pallas import tpu as pltpu
from jax.experimental.pallas import tpu_sc as plsc

_PAIRS_A = (0, 2, 4, 6, 8, 10, 12, 14, 16, 18, 20, 22, 24, 0, 1, 4)
_PAIRS_B = (1, 3, 5, 7, 9, 11, 13, 15, 17, 19, 21, 23, 25, 2, 3, 6)

_I = 1000
_D = 26
_B = 16384
_P = 16

_NC = 2
_NS = 16
_NW = _NC * _NS
_BW = _B // _NW
_GROUPS = _BW // 16
_QUARTERS = _BW // 128
_NROW = _P * _QUARTERS


def _sc_body(xT_hbm, t0_hbm, singles_hbm, pairs_hbm, out_hbm,
             xT_v, t0_v, singles_v, pidx_v, par_v, prow_v, out_v, sem):
    wid = lax.axis_index("s") * _NC + lax.axis_index("c")
    base = wid * _BW

    pltpu.sync_copy(xT_hbm.at[:, pl.ds(base, _BW)], xT_v)
    pltpu.sync_copy(t0_hbm, t0_v)
    pltpu.sync_copy(singles_hbm, singles_v)

    for p in range(_P):
        ra, rb = _PAIRS_A[p], _PAIRS_B[p]
        for q in range(_QUARTERS):
            row = p * _QUARTERS + q

            def build(c, _, row=row, ra=ra, rb=rb, q=q, p=p):
                b0 = q * 128 + c * 16
                ia = xT_v[ra, pl.ds(b0, 16)]
                ib = xT_v[rb, pl.ds(b0, 16)]
                e = p * (_I * _I) + ia * _I + ib
                pidx_v[row, pl.ds(c * 16, 16)] = e >> 1
                par_v[row, pl.ds(c * 16, 16)] = e & 1
                return 0

            lax.fori_loop(0, 8, build, 0)
            pltpu.make_async_copy(
                pairs_hbm.at[pidx_v.at[row]], prow_v.at[row], sem
            ).start()

    def singles_acc(g, _):
        b0 = g * 16
        acc = t0_v[...]
        for d in range(_D):
            xv = xT_v[d, pl.ds(b0, 16)]
            acc = acc + plsc.load_gather(singles_v, [xv + d * _I])
        out_v[pl.ds(b0, 16)] = acc
        return 0

    lax.fori_loop(0, _GROUPS, singles_acc, 0)

    def drain(j, _):
        pltpu.make_async_copy(
            pairs_hbm.at[pidx_v.at[j]], prow_v.at[j], sem
        ).wait()
        return 0

    lax.fori_loop(0, _NROW, drain, 0)

    mask_hi = jnp.full((16,), -65536, jnp.int32)
    zero = jnp.zeros((16,), jnp.int32)
    for j in range(_NROW):
        q = j % _QUARTERS

        def pairs_acc(c, _, j=j, q=q):
            sl = pl.ds(c * 16, 16)
            u = prow_v[j, sl]
            f_lo = plsc.bitcast(u << 16, jnp.float32)
            f_hi = plsc.bitcast(u & mask_hi, jnp.float32)
            f = jnp.where(par_v[j, sl] == zero, f_lo, f_hi)
            osl = pl.ds(q * 128 + c * 16, 16)
            out_v[osl] = out_v[osl] + f
            return 0

        lax.fori_loop(0, 8, pairs_acc, 0)

    pltpu.sync_copy(out_v, out_hbm.at[pl.ds(base, _BW)])


_sc_call = functools.partial(
    pl.kernel,
    mesh=plsc.VectorSubcoreMesh(core_axis_name="c", subcore_axis_name="s"),
    out_type=jax.ShapeDtypeStruct((_B,), jnp.float32),
    compiler_params=pltpu.CompilerParams(needs_layout_passes=False),
    scratch_types=[
        pltpu.VMEM((_D, _BW), jnp.int32),
        pltpu.VMEM((16,), jnp.float32),
        pltpu.VMEM((_D * _I,), jnp.float32),
        pltpu.VMEM((_NROW, 128), jnp.int32),
        pltpu.VMEM((_NROW, 128), jnp.int32),
        pltpu.VMEM((_NROW, 128), jnp.int32),
        pltpu.VMEM((_BW,), jnp.float32),
        pltpu.SemaphoreType.DMA,
    ],
)(_sc_body)


@jax.jit
def kernel(x, theta0, theta_singles, theta_pairs):
    xT = x.T.astype(jnp.int32)
    t0 = jnp.broadcast_to(theta0.astype(jnp.float32).reshape(1), (16,))
    singles = theta_singles.reshape(-1).astype(jnp.float32)
    pairs_w = lax.bitcast_convert_type(
        theta_pairs.astype(jnp.bfloat16).reshape(_P * _I * _I // 2, 2),
        jnp.int32,
    )
    return _sc_call(xT, t0, singles, pairs_w)


# speedup vs baseline: 21.2477x; 21.2477x over previous
import functools

import jax
import jax.numpy as jnp
from jax import lax
from jax.experimental import pallas as pl
from jax.experimental.pallas import tpu as pltpu
from jax.experimental.pallas import tpu_sc as plsc

_PAIRS_A = (0, 2, 4, 6, 8, 10, 12, 14, 16, 18, 20, 22, 24, 0, 1, 4)
_PAIRS_B = (1, 3, 5, 7, 9, 11, 13, 15, 17, 19, 21, 23, 25, 2, 3, 6)

_I = 1000
_D = 26
_B = 16384
_P = 16

_NC = 2
_NS = 16
_NW = _NC * _NS
_BW = _B // _NW
_GROUPS = _BW // 16
_QUARTERS = _BW // 128
_NROW = _P * _QUARTERS


def _sc_body(xT_hbm, t0_hbm, singles_hbm, pairs_hbm, out_hbm,
             xT_v, t0_v, singles_v, pidx_v, par_v, prow_v, out_v, sem):
    wid = lax.axis_index("s") * _NC + lax.axis_index("c")
    base = wid * _BW

    pltpu.sync_copy(xT_hbm.at[:, pl.ds(base, _BW)], xT_v)
    pltpu.sync_copy(t0_hbm, t0_v)
    pltpu.sync_copy(singles_hbm, singles_v)

    for p in range(_P):
        ra, rb = _PAIRS_A[p], _PAIRS_B[p]
        for q in range(_QUARTERS):
            row = p * _QUARTERS + q

            def build(c, _, row=row, ra=ra, rb=rb, q=q, p=p):
                b0 = q * 128 + c * 16
                ia = xT_v[ra, pl.ds(b0, 16)]
                ib = xT_v[rb, pl.ds(b0, 16)]
                hi = (ib >= _I // 2).astype(jnp.int32)
                jm = ib - hi * (_I // 2)
                pidx_v[row, pl.ds(c * 16, 16)] = (
                    p * (_I * _I // 2) + ia * (_I // 2) + jm
                )
                par_v[row, pl.ds(c * 16, 16)] = hi
                return 0

            lax.fori_loop(0, 8, build, 0)
            pltpu.make_async_copy(
                pairs_hbm.at[pidx_v.at[row]], prow_v.at[row], sem
            ).start()

    def singles_acc(g, _):
        b0 = g * 16
        acc = t0_v[...]
        for d in range(_D):
            xv = xT_v[d, pl.ds(b0, 16)]
            acc = acc + plsc.load_gather(singles_v, [xv + d * _I])
        out_v[pl.ds(b0, 16)] = acc
        return 0

    lax.fori_loop(0, _GROUPS, singles_acc, 0)

    def drain(j, _):
        pltpu.make_async_copy(
            pairs_hbm.at[pidx_v.at[j]], prow_v.at[j], sem
        ).wait()
        return 0

    lax.fori_loop(0, _NROW, drain, 0)

    mask_hi = jnp.full((16,), -65536, jnp.int32)
    zero = jnp.zeros((16,), jnp.int32)
    for j in range(_NROW):
        q = j % _QUARTERS

        def pairs_acc(c, _, j=j, q=q):
            sl = pl.ds(c * 16, 16)
            u = prow_v[j, sl]
            f_lo = plsc.bitcast(u << 16, jnp.float32)
            f_hi = plsc.bitcast(u & mask_hi, jnp.float32)
            f = jnp.where(par_v[j, sl] == zero, f_lo, f_hi)
            osl = pl.ds(q * 128 + c * 16, 16)
            out_v[osl] = out_v[osl] + f
            return 0

        lax.fori_loop(0, 8, pairs_acc, 0)

    pltpu.sync_copy(out_v, out_hbm.at[pl.ds(base, _BW)])


_sc_call = functools.partial(
    pl.kernel,
    mesh=plsc.VectorSubcoreMesh(core_axis_name="c", subcore_axis_name="s"),
    out_type=jax.ShapeDtypeStruct((_B,), jnp.float32),
    compiler_params=pltpu.CompilerParams(needs_layout_passes=False),
    scratch_types=[
        pltpu.VMEM((_D, _BW), jnp.int32),
        pltpu.VMEM((16,), jnp.float32),
        pltpu.VMEM((_D * _I,), jnp.float32),
        pltpu.VMEM((_NROW, 128), jnp.int32),
        pltpu.VMEM((_NROW, 128), jnp.int32),
        pltpu.VMEM((_NROW, 128), jnp.int32),
        pltpu.VMEM((_BW,), jnp.float32),
        pltpu.SemaphoreType.DMA,
    ],
)(_sc_body)


@jax.jit
def kernel(x, theta0, theta_singles, theta_pairs):
    xT = x.T.astype(jnp.int32)
    t0 = jnp.broadcast_to(theta0.astype(jnp.float32).reshape(1), (16,))
    singles = theta_singles.reshape(-1).astype(jnp.float32)
    u = lax.bitcast_convert_type(theta_pairs.astype(jnp.float32), jnp.uint32)
    r = (u + 0x7FFF + ((u >> 16) & 1)) >> 16
    w = r[:, :, : _I // 2] | (r[:, :, _I // 2 :] << 16)
    pairs_w = lax.bitcast_convert_type(w, jnp.int32).reshape(-1)
    return _sc_call(xT, t0, singles, pairs_w)


# speedup vs baseline: 22.8643x; 1.0761x over previous
import functools

import jax
import jax.numpy as jnp
from jax import lax
from jax.experimental import pallas as pl
from jax.experimental.pallas import tpu as pltpu
from jax.experimental.pallas import tpu_sc as plsc

_PAIRS_A = (0, 2, 4, 6, 8, 10, 12, 14, 16, 18, 20, 22, 24, 0, 1, 4)
_PAIRS_B = (1, 3, 5, 7, 9, 11, 13, 15, 17, 19, 21, 23, 25, 2, 3, 6)

_I = 1000
_D = 26
_B = 16384
_P = 16

_JT = 8
_IT = _I // 8
_WPP = _JT * _IT * 1024

_NC = 2
_NS = 16
_NW = _NC * _NS
_BW = _B // _NW
_GROUPS = _BW // 16
_QUARTERS = _BW // 128
_NROW = _P * _QUARTERS


def _untile_body(x_ref, o_ref):
    o_ref[...] = x_ref[0].reshape(_I * 128)


_untile = pl.pallas_call(
    _untile_body,
    grid=(_P, _JT),
    in_specs=[pl.BlockSpec((1, _I, 128), lambda p, jt: (p, 0, jt))],
    out_specs=pl.BlockSpec((_I * 128,), lambda p, jt: (p * _JT + jt,)),
    out_shape=jax.ShapeDtypeStruct((_P * _WPP,), jnp.float32),
)


def _sc_body(xT_hbm, t0_hbm, singles_hbm, pairs_hbm, out_hbm,
             xT_v, t0_v, singles_v, pidx_v, prow_v, out_v, sem):
    wid = lax.axis_index("s") * _NC + lax.axis_index("c")
    base = wid * _BW

    pltpu.sync_copy(xT_hbm.at[:, pl.ds(base, _BW)], xT_v)
    pltpu.sync_copy(t0_hbm, t0_v)
    pltpu.sync_copy(singles_hbm, singles_v)

    for p in range(_P):
        ra, rb = _PAIRS_A[p], _PAIRS_B[p]
        for q in range(_QUARTERS):
            row = p * _QUARTERS + q

            def build(c, _, row=row, ra=ra, rb=rb, q=q, p=p):
                b0 = q * 128 + c * 16
                ia = xT_v[ra, pl.ds(b0, 16)]
                ib = xT_v[rb, pl.ds(b0, 16)]
                pidx_v[row, pl.ds(c * 16, 16)] = (
                    p * _WPP
                    + (ib >> 7) * (_IT * 1024)
                    + (ia >> 3) * 1024
                    + (ia & 7) * 128
                    + (ib & 127)
                )
                return 0

            lax.fori_loop(0, 8, build, 0)
            pltpu.make_async_copy(
                pairs_hbm.at[pidx_v.at[row]], prow_v.at[row], sem
            ).start()

    def singles_acc(g, _):
        b0 = g * 16
        acc = t0_v[...]
        for d in range(_D):
            xv = xT_v[d, pl.ds(b0, 16)]
            acc = acc + plsc.load_gather(singles_v, [xv + d * _I])
        out_v[pl.ds(b0, 16)] = acc
        return 0

    lax.fori_loop(0, _GROUPS, singles_acc, 0)

    def drain(j, _):
        pltpu.make_async_copy(pairs_hbm.at[pidx_v.at[j]], prow_v.at[j], sem).wait()
        return 0

    lax.fori_loop(0, _NROW, drain, 0)

    for j in range(_NROW):
        q = j % _QUARTERS

        def pairs_acc(c, _, j=j, q=q):
            sl = pl.ds(q * 128 + c * 16, 16)
            out_v[sl] = out_v[sl] + prow_v[j, pl.ds(c * 16, 16)]
            return 0

        lax.fori_loop(0, 8, pairs_acc, 0)

    pltpu.sync_copy(out_v, out_hbm.at[pl.ds(base, _BW)])


_sc_call = functools.partial(
    pl.kernel,
    mesh=plsc.VectorSubcoreMesh(core_axis_name="c", subcore_axis_name="s"),
    out_type=jax.ShapeDtypeStruct((_B,), jnp.float32),
    compiler_params=pltpu.CompilerParams(needs_layout_passes=False),
    scratch_types=[
        pltpu.VMEM((_D, _BW), jnp.int32),
        pltpu.VMEM((16,), jnp.float32),
        pltpu.VMEM((_D * _I,), jnp.float32),
        pltpu.VMEM((_NROW, 128), jnp.int32),
        pltpu.VMEM((_NROW, 128), jnp.float32),
        pltpu.VMEM((_BW,), jnp.float32),
        pltpu.SemaphoreType.DMA,
    ],
)(_sc_body)


@jax.jit
def kernel(x, theta0, theta_singles, theta_pairs):
    xT = x.T.astype(jnp.int32)
    t0 = jnp.broadcast_to(theta0.astype(jnp.float32).reshape(1), (16,))
    singles = theta_singles.reshape(-1).astype(jnp.float32)
    pairs_lin = _untile(theta_pairs.astype(jnp.float32))
    return _sc_call(xT, t0, singles, pairs_lin)


# speedup vs baseline: 40.7152x; 1.7807x over previous
import functools

import jax
import jax.numpy as jnp
from jax import lax
from jax.experimental import pallas as pl
from jax.experimental.pallas import tpu as pltpu
from jax.experimental.pallas import tpu_sc as plsc

_PAIRS_A = (0, 2, 4, 6, 8, 10, 12, 14, 16, 18, 20, 22, 24, 0, 1, 4)
_PAIRS_B = (1, 3, 5, 7, 9, 11, 13, 15, 17, 19, 21, 23, 25, 2, 3, 6)

_I = 1000
_D = 26
_B = 16384
_P = 16

_JT = 8
_IT = _I // 8
_WPP = _JT * _IT * 1024

_NC = 2
_NS = 16
_NW = _NC * _NS
_BW = _B // _NW
_GROUPS = _BW // 16
_QUARTERS = _BW // 128
_NROW = _P * _QUARTERS


def _untile_body(x_ref, o_ref):
    for jt in range(_JT - 1):
        o_ref[pl.ds(jt * _I, _I), :] = x_ref[0, :, pl.ds(jt * 128, 128)]
    rem = _I - (_JT - 1) * 128
    o_ref[pl.ds((_JT - 1) * _I, _I), pl.ds(0, rem)] = x_ref[
        0, :, pl.ds((_JT - 1) * 128, rem)
    ]


_untile = pl.pallas_call(
    _untile_body,
    grid=(_P,),
    in_specs=[pl.BlockSpec((1, _I, _I), lambda p: (p, 0, 0))],
    out_specs=pl.BlockSpec((_JT * _I, 128), lambda p: (p, 0)),
    out_shape=jax.ShapeDtypeStruct((_P * _JT * _I, 128), jnp.float32),
)


def _sc_body(xT_hbm, t0_hbm, singles_hbm, pairs_hbm, out_hbm,
             xT_v, t0_v, singles_v, pidx_v, prow_v, out_v, sem):
    wid = lax.axis_index("s") * _NC + lax.axis_index("c")
    base = wid * _BW

    pltpu.sync_copy(xT_hbm.at[:, pl.ds(base, _BW)], xT_v)
    pltpu.sync_copy(t0_hbm, t0_v)
    pltpu.sync_copy(singles_hbm, singles_v)

    for p in range(_P):
        ra, rb = _PAIRS_A[p], _PAIRS_B[p]
        for q in range(_QUARTERS):
            row = p * _QUARTERS + q

            def build(c, _, row=row, ra=ra, rb=rb, q=q, p=p):
                b0 = q * 128 + c * 16
                ia = xT_v[ra, pl.ds(b0, 16)]
                ib = xT_v[rb, pl.ds(b0, 16)]
                pidx_v[row, pl.ds(c * 16, 16)] = (
                    (p * _JT + (ib >> 7)) * (_I * 128)
                    + ia * 128
                    + (ib & 127)
                )
                return 0

            lax.fori_loop(0, 8, build, 0)
            pltpu.make_async_copy(
                pairs_hbm.at[pidx_v.at[row]], prow_v.at[row], sem
            ).start()

    def singles_acc(g, _):
        b0 = g * 16
        acc = t0_v[...]
        for d in range(_D):
            xv = xT_v[d, pl.ds(b0, 16)]
            acc = acc + plsc.load_gather(singles_v, [xv + d * _I])
        out_v[pl.ds(b0, 16)] = acc
        return 0

    lax.fori_loop(0, _GROUPS, singles_acc, 0)

    def drain(j, _):
        pltpu.make_async_copy(pairs_hbm.at[pidx_v.at[j]], prow_v.at[j], sem).wait()
        return 0

    lax.fori_loop(0, _NROW, drain, 0)

    for j in range(_NROW):
        q = j % _QUARTERS

        def pairs_acc(c, _, j=j, q=q):
            sl = pl.ds(q * 128 + c * 16, 16)
            out_v[sl] = out_v[sl] + prow_v[j, pl.ds(c * 16, 16)]
            return 0

        lax.fori_loop(0, 8, pairs_acc, 0)

    pltpu.sync_copy(out_v, out_hbm.at[pl.ds(base, _BW)])


_sc_call = functools.partial(
    pl.kernel,
    mesh=plsc.VectorSubcoreMesh(core_axis_name="c", subcore_axis_name="s"),
    out_type=jax.ShapeDtypeStruct((_B,), jnp.float32),
    compiler_params=pltpu.CompilerParams(needs_layout_passes=False),
    scratch_types=[
        pltpu.VMEM((_D, _BW), jnp.int32),
        pltpu.VMEM((16,), jnp.float32),
        pltpu.VMEM((_D * _I,), jnp.float32),
        pltpu.VMEM((_NROW, 128), jnp.int32),
        pltpu.VMEM((_NROW, 128), jnp.float32),
        pltpu.VMEM((_BW,), jnp.float32),
        pltpu.SemaphoreType.DMA,
    ],
)(_sc_body)


@jax.jit
def kernel(x, theta0, theta_singles, theta_pairs):
    xT = x.T.astype(jnp.int32)
    t0 = jnp.broadcast_to(theta0.astype(jnp.float32).reshape(1), (16,))
    singles = theta_singles.reshape(-1).astype(jnp.float32)
    pairs_lin = _untile(theta_pairs.astype(jnp.float32)).reshape(-1)
    return _sc_call(xT, t0, singles, pairs_lin)


# speedup vs baseline: 46.1934x; 1.1345x over previous
import functools

import jax
import jax.numpy as jnp
from jax import lax
from jax.experimental import pallas as pl
from jax.experimental.pallas import tpu as pltpu
from jax.experimental.pallas import tpu_sc as plsc

_PAIRS_A = (0, 2, 4, 6, 8, 10, 12, 14, 16, 18, 20, 22, 24, 0, 1, 4)
_PAIRS_B = (1, 3, 5, 7, 9, 11, 13, 15, 17, 19, 21, 23, 25, 2, 3, 6)

_I = 1000
_D = 26
_B = 16384
_P = 16

_JT = 8
_IT = _I // 8
_WPP = _JT * _IT * 1024

_NC = 2
_NS = 16
_NW = _NC * _NS
_BW = _B // _NW
_GROUPS = _BW // 16
_QUARTERS = _BW // 128
_NROW = _P * _QUARTERS


def _rne_bf16_bits(v):
    u = lax.bitcast_convert_type(v, jnp.uint32)
    return (u + 0x7FFF + ((u >> 16) & 1)) >> 16


def _untile_body(x1_ref, x2_ref, o_ref):
    for jt in range(_JT):
        w = 128 if jt < _JT - 1 else _I - (_JT - 1) * 128
        a = _rne_bf16_bits(x1_ref[0, :, pl.ds(jt * 128, w)])
        b = _rne_bf16_bits(x2_ref[0, :, pl.ds(jt * 128, w)])
        o_ref[pl.ds(jt * _I, _I), pl.ds(0, w)] = lax.bitcast_convert_type(
            a | (b << 16), jnp.int32
        )


_untile = pl.pallas_call(
    _untile_body,
    grid=(_P // 2,),
    in_specs=[
        pl.BlockSpec((1, _I, _I), lambda g: (g, 0, 0)),
        pl.BlockSpec((1, _I, _I), lambda g: (g + _P // 2, 0, 0)),
    ],
    out_specs=pl.BlockSpec((_JT * _I, 128), lambda g: (g, 0)),
    out_shape=jax.ShapeDtypeStruct((_P // 2 * _JT * _I, 128), jnp.int32),
)


def _sc_body(xT_hbm, t0_hbm, singles_hbm, pairs_hbm, out_hbm,
             xT_v, t0_v, singles_v, pidx_v, prow_v, out_v, sem):
    wid = lax.axis_index("s") * _NC + lax.axis_index("c")
    base = wid * _BW

    pltpu.sync_copy(xT_hbm.at[:, pl.ds(base, _BW)], xT_v)
    pltpu.sync_copy(t0_hbm, t0_v)
    pltpu.sync_copy(singles_hbm, singles_v)

    for p in range(_P):
        ra, rb = _PAIRS_A[p], _PAIRS_B[p]
        for q in range(_QUARTERS):
            row = p * _QUARTERS + q

            def build(c, _, row=row, ra=ra, rb=rb, q=q, p=p):
                b0 = q * 128 + c * 16
                ia = xT_v[ra, pl.ds(b0, 16)]
                ib = xT_v[rb, pl.ds(b0, 16)]
                pidx_v[row, pl.ds(c * 16, 16)] = (
                    ((p % (_P // 2)) * _JT + (ib >> 7)) * (_I * 128)
                    + ia * 128
                    + (ib & 127)
                )
                return 0

            lax.fori_loop(0, 8, build, 0)
            pltpu.make_async_copy(
                pairs_hbm.at[pidx_v.at[row]], prow_v.at[row], sem
            ).start()

    def singles_acc(g, _):
        b0 = g * 16
        acc = t0_v[...]
        for d in range(_D):
            xv = xT_v[d, pl.ds(b0, 16)]
            acc = acc + plsc.load_gather(singles_v, [xv + d * _I])
        out_v[pl.ds(b0, 16)] = acc
        return 0

    lax.fori_loop(0, _GROUPS, singles_acc, 0)

    def drain(j, _):
        pltpu.make_async_copy(pairs_hbm.at[pidx_v.at[j]], prow_v.at[j], sem).wait()
        return 0

    lax.fori_loop(0, _NROW, drain, 0)

    mask_hi = jnp.full((16,), -65536, jnp.int32)
    for j in range(_NROW):
        q = j % _QUARTERS
        high = (j >> 2) >= _P // 2

        def pairs_acc(c, _, j=j, q=q, high=high):
            sl = pl.ds(q * 128 + c * 16, 16)
            u = prow_v[j, pl.ds(c * 16, 16)]
            if high:
                f = plsc.bitcast(u & mask_hi, jnp.float32)
            else:
                f = plsc.bitcast(u << 16, jnp.float32)
            out_v[sl] = out_v[sl] + f
            return 0

        lax.fori_loop(0, 8, pairs_acc, 0)

    pltpu.sync_copy(out_v, out_hbm.at[pl.ds(base, _BW)])


_sc_call = functools.partial(
    pl.kernel,
    mesh=plsc.VectorSubcoreMesh(core_axis_name="c", subcore_axis_name="s"),
    out_type=jax.ShapeDtypeStruct((_B,), jnp.float32),
    compiler_params=pltpu.CompilerParams(needs_layout_passes=False),
    scratch_types=[
        pltpu.VMEM((_D, _BW), jnp.int32),
        pltpu.VMEM((16,), jnp.float32),
        pltpu.VMEM((_D * _I,), jnp.float32),
        pltpu.VMEM((_NROW, 128), jnp.int32),
        pltpu.VMEM((_NROW, 128), jnp.int32),
        pltpu.VMEM((_BW,), jnp.float32),
        pltpu.SemaphoreType.DMA,
    ],
)(_sc_body)


@jax.jit
def kernel(x, theta0, theta_singles, theta_pairs):
    xT = x.T.astype(jnp.int32)
    t0 = jnp.broadcast_to(theta0.astype(jnp.float32).reshape(1), (16,))
    singles = theta_singles.reshape(-1).astype(jnp.float32)
    tp = theta_pairs.astype(jnp.float32)
    pairs_lin = _untile(tp, tp).reshape(-1)
    return _sc_call(xT, t0, singles, pairs_lin)


# speedup vs baseline: 47.4543x; 1.0273x over previous
import functools

import jax
import jax.numpy as jnp
from jax import lax
from jax.experimental import pallas as pl
from jax.experimental.pallas import tpu as pltpu
from jax.experimental.pallas import tpu_sc as plsc

_PAIRS_A = (0, 2, 4, 6, 8, 10, 12, 14, 16, 18, 20, 22, 24, 0, 1, 4)
_PAIRS_B = (1, 3, 5, 7, 9, 11, 13, 15, 17, 19, 21, 23, 25, 2, 3, 6)

_I = 1000
_D = 26
_B = 16384
_P = 16

_JT = 8

_NC = 2
_NS = 16
_NW = _NC * _NS
_BW = _B // _NW
_GROUPS = _BW // 16
_QUARTERS = _BW // 128
_NROW = _P * _QUARTERS


def _rne_bf16_bits(v):
    u = lax.bitcast_convert_type(v, jnp.uint32)
    return (u + 0x7FFF + ((u >> 16) & 1)) >> 16


def _untile_body(x1_ref, x2_ref, o_ref):
    for jt in range(_JT):
        w = 128 if jt < _JT - 1 else _I - (_JT - 1) * 128
        a = _rne_bf16_bits(x1_ref[0, :, pl.ds(jt * 128, w)])
        b = _rne_bf16_bits(x2_ref[0, :, pl.ds(jt * 128, w)])
        o_ref[pl.ds(jt * _I, _I), pl.ds(0, w)] = lax.bitcast_convert_type(
            a | (b << 16), jnp.int32
        )


_untile = pl.pallas_call(
    _untile_body,
    grid=(_P // 2,),
    in_specs=[
        pl.BlockSpec((1, _I, _I), lambda g: (g, 0, 0)),
        pl.BlockSpec((1, _I, _I), lambda g: (g + _P // 2, 0, 0)),
    ],
    out_specs=pl.BlockSpec((_JT * _I, 128), lambda g: (g, 0)),
    out_shape=jax.ShapeDtypeStruct((_P // 2 * _JT * _I, 128), jnp.int32),
)


def _sc1_body(xT_hbm, t0_hbm, singles_hbm, pidx_hbm, part_hbm,
              xT_v, t0_v, singles_v, pidx_v, out_v):
    wid = lax.axis_index("s") * _NC + lax.axis_index("c")
    base = wid * _BW

    pltpu.sync_copy(xT_hbm.at[:, pl.ds(base, _BW)], xT_v)
    pltpu.sync_copy(t0_hbm, t0_v)
    pltpu.sync_copy(singles_hbm, singles_v)

    for p in range(_P):
        ra, rb = _PAIRS_A[p], _PAIRS_B[p]
        for q in range(_QUARTERS):
            row = p * _QUARTERS + q

            def build(c, _, row=row, ra=ra, rb=rb, q=q, p=p):
                b0 = q * 128 + c * 16
                ia = xT_v[ra, pl.ds(b0, 16)]
                ib = xT_v[rb, pl.ds(b0, 16)]
                pidx_v[row, pl.ds(c * 16, 16)] = (
                    ((p % (_P // 2)) * _JT + (ib >> 7)) * (_I * 128)
                    + ia * 128
                    + (ib & 127)
                )
                return 0

            lax.fori_loop(0, 8, build, 0)

    def singles_acc(g, _):
        b0 = g * 16
        acc = t0_v[...]
        for d in range(_D):
            xv = xT_v[d, pl.ds(b0, 16)]
            acc = acc + plsc.load_gather(singles_v, [xv + d * _I])
        out_v[pl.ds(b0, 16)] = acc
        return 0

    lax.fori_loop(0, _GROUPS, singles_acc, 0)

    pltpu.sync_copy(pidx_v, pidx_hbm.at[wid])
    pltpu.sync_copy(out_v, part_hbm.at[pl.ds(base, _BW)])


_sc1_call = functools.partial(
    pl.kernel,
    mesh=plsc.VectorSubcoreMesh(core_axis_name="c", subcore_axis_name="s"),
    out_type=(
        jax.ShapeDtypeStruct((_NW, _NROW, 128), jnp.int32),
        jax.ShapeDtypeStruct((_B,), jnp.float32),
    ),
    compiler_params=pltpu.CompilerParams(needs_layout_passes=False),
    scratch_types=[
        pltpu.VMEM((_D, _BW), jnp.int32),
        pltpu.VMEM((16,), jnp.float32),
        pltpu.VMEM((_D * _I,), jnp.float32),
        pltpu.VMEM((_NROW, 128), jnp.int32),
        pltpu.VMEM((_BW,), jnp.float32),
    ],
)(_sc1_body)


def _sc2_body(pairs_hbm, pidx_hbm, part_hbm, out_hbm,
              pidx_v, prow_v, out_v, sem):
    wid = lax.axis_index("s") * _NC + lax.axis_index("c")
    base = wid * _BW

    pltpu.sync_copy(pidx_hbm.at[wid], pidx_v)

    def fire(j, _):
        pltpu.make_async_copy(pairs_hbm.at[pidx_v.at[j]], prow_v.at[j], sem).start()
        return 0

    lax.fori_loop(0, _NROW, fire, 0)

    pltpu.sync_copy(part_hbm.at[pl.ds(base, _BW)], out_v)

    def drain(j, _):
        pltpu.make_async_copy(pairs_hbm.at[pidx_v.at[j]], prow_v.at[j], sem).wait()
        return 0

    lax.fori_loop(0, _NROW, drain, 0)

    mask_hi = jnp.full((16,), -65536, jnp.int32)
    for j in range(_NROW):
        q = j % _QUARTERS
        high = (j >> 2) >= _P // 2

        def pairs_acc(c, _, j=j, q=q, high=high):
            sl = pl.ds(q * 128 + c * 16, 16)
            u = prow_v[j, pl.ds(c * 16, 16)]
            if high:
                f = plsc.bitcast(u & mask_hi, jnp.float32)
            else:
                f = plsc.bitcast(u << 16, jnp.float32)
            out_v[sl] = out_v[sl] + f
            return 0

        lax.fori_loop(0, 8, pairs_acc, 0)

    pltpu.sync_copy(out_v, out_hbm.at[pl.ds(base, _BW)])


_sc2_call = functools.partial(
    pl.kernel,
    mesh=plsc.VectorSubcoreMesh(core_axis_name="c", subcore_axis_name="s"),
    out_type=jax.ShapeDtypeStruct((_B,), jnp.float32),
    compiler_params=pltpu.CompilerParams(needs_layout_passes=False),
    scratch_types=[
        pltpu.VMEM((_NROW, 128), jnp.int32),
        pltpu.VMEM((_NROW, 128), jnp.int32),
        pltpu.VMEM((_BW,), jnp.float32),
        pltpu.SemaphoreType.DMA,
    ],
)(_sc2_body)


@jax.jit
def kernel(x, theta0, theta_singles, theta_pairs):
    xT = x.T.astype(jnp.int32)
    t0 = jnp.broadcast_to(theta0.astype(jnp.float32).reshape(1), (16,))
    singles = theta_singles.reshape(-1).astype(jnp.float32)
    tp = theta_pairs.astype(jnp.float32)
    pairs_lin = _untile(tp, tp).reshape(-1)
    pidx, part = _sc1_call(xT, t0, singles)
    return _sc2_call(pairs_lin, pidx, part)
